# Initial kernel scaffold; baseline (speedup 1.0000x reference)
import jax
import jax.numpy as jnp
from jax.experimental import pallas as pl


def kernel(input_ids):
    raise NotImplementedError("write your pallas kernel here")


# speedup vs baseline: 1.6386x; 1.6386x over previous
import functools

import numpy as np
import jax
import jax.numpy as jnp
from jax import lax
from jax.experimental import pallas as pl
from jax.experimental.pallas import tpu as pltpu
from jax.experimental.pallas import tpu_sc as plsc

jax.config.update("jax_enable_x64", True)

_MAX_NGRAM = 4
_N_HEAD = 8
_VOCAB_PER_NGRAM = [100000, 100000, 100000]
_TOKENIZER_VOCAB = 50000
_SEED = 42
_LAYER_ID = 0
_PRIME_1 = 10007


def _isprime(n):
    if n < 2:
        return False
    if n % 2 == 0:
        return n == 2
    i = 3
    while i * i <= n:
        if n % i == 0:
            return False
        i += 2
    return True


def _find_next_prime(start, seen):
    c = start + 1
    while True:
        if _isprime(c) and c not in seen:
            return c
        c += 1


def _build_params():
    max_long = np.iinfo(np.int64).max
    m_max = int(max_long // _TOKENIZER_VOCAB)
    half_bound = max(1, m_max // 2)
    g = np.random.default_rng(int(_SEED + _PRIME_1 * _LAYER_ID))
    r = g.integers(low=0, high=half_bound, size=(_MAX_NGRAM,), dtype=np.int64)
    multipliers = [int(x) * 2 + 1 for x in r]
    seen = set()
    primes = []
    for ngram in range(2, _MAX_NGRAM + 1):
        start = _VOCAB_PER_NGRAM[ngram - 2] - 1
        for _ in range(_N_HEAD):
            p = _find_next_prime(start, seen)
            seen.add(p)
            primes.append(p)
            start = p
    return multipliers, primes


_MULTIPLIERS, _PRIMES = _build_params()
_MLIMBS = [[(m >> (16 * j)) & 0xFFFF for j in range(4)] for m in _MULTIPLIERS]
_DTAB = [[(1 << (8 * i)) % p for i in range(8)] for p in _PRIMES]
_INVP = [float(np.float32(1.0) / np.float32(p)) for p in _PRIMES]

_NC, _NS, _LANES = 2, 16, 16
_NW = _NC * _NS


_I32 = jnp.int32


def _product_limbs(t, mk):
    mask16 = _I32(0xFFFF)
    sh16 = _I32(16)
    p = [t * _I32(mk[j]) for j in range(4)]
    lo = [pj & mask16 for pj in p]
    hi = [(pj >> sh16) & mask16 for pj in p]
    l0 = lo[0]
    l1 = hi[0] + lo[1]
    l2 = hi[1] + lo[2]
    l3 = hi[2] + lo[3]
    c = l1 >> sh16
    l1 = l1 & mask16
    l2 = l2 + c
    c = l2 >> sh16
    l2 = l2 & mask16
    l3 = (l3 + c) & mask16
    return [l0, l1, l2, l3]


def _mod_head(s, p, invp):
    q = (s.astype(jnp.float32) * jnp.float32(invp)).astype(jnp.int32)
    r = s - q * jnp.int32(p)
    r = jnp.where(r < 0, r + jnp.int32(p), r)
    r = jnp.where(r >= jnp.int32(p), r - jnp.int32(p), r)
    return r


@functools.lru_cache(maxsize=None)
def _make_sc_call(total_tokens, tokens_per_row):
    chunk = total_tokens // _NW
    groups = chunk // _LANES
    chunks_per_row = tokens_per_row // chunk
    n_hash = (_MAX_NGRAM - 1) * _N_HEAD

    mesh = plsc.VectorSubcoreMesh(core_axis_name="c", subcore_axis_name="s")

    def body(ids_hbm, out_hbm, tok_v, obuf_v):
        cid = lax.axis_index("c").astype(_I32)
        sid = lax.axis_index("s").astype(_I32)
        wid = cid * _I32(_NS) + sid
        base = wid * _I32(chunk)
        at_row_start = (wid % _I32(chunks_per_row)) == _I32(0)

        @pl.when(at_row_start)
        def _():
            tok_v[pl.ds(0, _LANES)] = jnp.zeros((_LANES,), jnp.int32)
            pltpu.sync_copy(ids_hbm.at[pl.ds(base, chunk)],
                            tok_v.at[pl.ds(16, chunk)])

        @pl.when(jnp.logical_not(at_row_start))
        def _():
            pltpu.sync_copy(ids_hbm.at[pl.ds(base - _I32(8), chunk + 8)],
                            tok_v.at[pl.ds(8, chunk + 8)])

        iota = lax.iota(jnp.int32, _LANES)

        def group(g, carry):
            lb = _I32(16) + g * _I32(_LANES)
            t0 = tok_v[pl.ds(lb, _LANES)]
            t1 = plsc.load_gather(tok_v, [iota + (lb - _I32(1))])
            t2 = plsc.load_gather(tok_v, [iota + (lb - _I32(2))])
            t3 = plsc.load_gather(tok_v, [iota + (lb - _I32(3))])

            prods = [_product_limbs(t, _MLIMBS[k])
                     for k, t in enumerate((t0, t1, t2, t3))]

            mask8 = _I32(0xFF)
            sh8 = _I32(8)
            sidx = iota * _I32(n_hash) + g * _I32(_LANES * n_hash)
            mix = prods[0]
            h = 0
            for n in range(1, _MAX_NGRAM):
                mix = [mix[j] ^ prods[n][j] for j in range(4)]
                bts = []
                for j in range(4):
                    bts.append(mix[j] & mask8)
                    bts.append(mix[j] >> sh8)
                for _ in range(_N_HEAD):
                    d = _DTAB[h]
                    s = bts[0] * _I32(d[0])
                    for i in range(1, 8):
                        s = s + bts[i] * _I32(d[i])
                    r = _mod_head(s, _PRIMES[h], _INVP[h])
                    plsc.store_scatter(obuf_v, [sidx + _I32(h)], r)
                    h += 1
            return carry

        lax.fori_loop(_I32(0), _I32(groups), group, _I32(0))
        pltpu.sync_copy(obuf_v, out_hbm.at[pl.ds(base * _I32(n_hash), chunk * n_hash)])

    return pl.kernel(
        body,
        out_type=jax.ShapeDtypeStruct((total_tokens * n_hash,), jnp.int32),
        mesh=mesh,
        compiler_params=pltpu.CompilerParams(needs_layout_passes=False),
        scratch_types=[
            pltpu.VMEM((16 + chunk,), jnp.int32),
            pltpu.VMEM((chunk * n_hash,), jnp.int32),
        ],
    )


def kernel(input_ids):
    b, t = input_ids.shape
    total = b * t
    assert total % _NW == 0 and (total // _NW) % _LANES == 0
    assert t % (total // _NW) == 0, "chunks must not straddle rows"
    ids32 = input_ids.astype(jnp.int32).reshape(total)
    out_flat = _make_sc_call(total, t)(ids32)
    n_hash = (_MAX_NGRAM - 1) * _N_HEAD
    return out_flat.reshape(b, t, n_hash).astype(jnp.int64)
